# baseline (speedup 1.0000x reference)
import functools

import jax
import jax.numpy as jnp
from jax import lax
from jax.experimental import pallas as pl
from jax.experimental.pallas import tpu as pltpu
from jax.experimental.pallas import tpu_sc as plsc

N = 10000
E = 320000
DIMS = [128, 64, 32, 16]
BN_EPS = 1e-5

NW = 32
C = 128
NPAIR = 160
EPAD = 16 * NPAIR * C
KSTAGE = 8
NC0S = {128: 120, 64: 112, 32: 104, 16: 88}
R = 10112
RPT = R // 16
CW = 1


def _make_agg(fo, with_counts):
    nc0 = NC0S[fo]
    mesh = plsc.VectorSubcoreMesh(
        core_axis_name="c", subcore_axis_name="s", num_cores=2, num_subcores=16
    )
    out_type = [jax.ShapeDtypeStruct((2, R, fo), jnp.float32)]
    scratch = [
        pltpu.VMEM_SHARED((R, fo), jnp.float32),
        pltpu.VMEM((KSTAGE, C), jnp.int32),
        pltpu.VMEM((KSTAGE, C), jnp.int32),
        pltpu.VMEM((C, fo), jnp.float32),
        pltpu.VMEM((C, fo), jnp.float32),
        pltpu.SemaphoreType.DMA,
        pltpu.SemaphoreType.DMA,
    ]
    if with_counts:
        out_type.append(jax.ShapeDtypeStruct((2, R, CW), jnp.float32))
        scratch += [
            pltpu.VMEM_SHARED((R, CW), jnp.float32),
            pltpu.VMEM((C, CW), jnp.float32),
        ]

    def body(table, srcb, dstb, zrows, *rest):
        if with_counts:
            (zcnt, onesb, outacc, outcnt, acc_sh, src_v, dst_v,
             rows0, rows1, gs0, gs1, cnt_sh, ones_v) = rest
        else:
            (outacc, acc_sh, src_v, dst_v,
             rows0, rows1, gs0, gs1) = rest
        rows = (rows0, rows1)
        gsem = (gs0, gs1)
        cid = lax.axis_index("c")
        sid = lax.axis_index("s")
        base = jnp.where(cid == 0, 0, nc0)
        nstage = jnp.where(cid == 0, nc0 // KSTAGE, (NPAIR - nc0) // KSTAGE)
        rs = pl.ds(sid * RPT, RPT)
        pltpu.sync_copy(zrows.at[rs], acc_sh.at[rs])
        if with_counts:
            pltpu.sync_copy(zcnt.at[rs], cnt_sh.at[rs])
            pltpu.sync_copy(onesb, ones_v)
        plsc.subcore_barrier()

        def stage(t, carry):
            pltpu.sync_copy(srcb.at[sid, pl.ds(base + t * KSTAGE, KSTAGE)], src_v)
            pltpu.sync_copy(dstb.at[sid, pl.ds(base + t * KSTAGE, KSTAGE)], dst_v)
            gd = [None, None]
            gd[0] = pltpu.async_copy(table.at[src_v.at[0]], rows[0], gsem[0])
            for j in range(KSTAGE):
                p = j % 2
                gd[p].wait()
                if j + 1 < KSTAGE:
                    gd[1 - p] = pltpu.async_copy(
                        table.at[src_v.at[j + 1]], rows[1 - p], gsem[1 - p])
                pltpu.sync_copy(rows[p], acc_sh.at[dst_v.at[j]], add=True)
                if with_counts:
                    pltpu.sync_copy(ones_v, cnt_sh.at[dst_v.at[j]], add=True)
            return carry

        lax.fori_loop(0, nstage, stage, 0)
        plsc.subcore_barrier()
        pltpu.sync_copy(acc_sh.at[rs], outacc.at[cid, rs])
        if with_counts:
            pltpu.sync_copy(cnt_sh.at[rs], outcnt.at[cid, rs])

    return pl.kernel(
        body, out_type=out_type, mesh=mesh, scratch_types=scratch,
        compiler_params=pltpu.CompilerParams(use_tc_tiling_on_sc=False))


def _dot(a, b):
    return jnp.dot(a, b, preferred_element_type=jnp.float32,
                   precision=lax.Precision.HIGHEST)


def _tc_pre(x, wl, wr, bl):
    def body(x_ref, wl_ref, wr_ref, bl_ref, y_ref, z_ref):
        xv = x_ref[...]
        y_ref[...] = _dot(xv, wl_ref[...])
        z_ref[...] = _dot(xv, wr_ref[...]) + bl_ref[...]

    fo = wl.shape[1]
    return pl.pallas_call(
        body,
        out_shape=[jax.ShapeDtypeStruct((N, fo), jnp.float32),
                   jax.ShapeDtypeStruct((N, fo), jnp.float32)],
    )(x, wl, wr, bl)


def _bn_relu(pre, g, b):
    mu = jnp.mean(pre, axis=0, keepdims=True)
    var = jnp.mean((pre - mu) ** 2, axis=0, keepdims=True)
    return jnp.maximum((pre - mu) * lax.rsqrt(var + BN_EPS) * g + b, 0.0)


def _tc_combine_first(acc, cnta, z, g, b):
    fo = z.shape[1]

    def body(acc_ref, cnta_ref, z_ref, g_ref, b_ref, h_ref, cnt_ref):
        cnt = jnp.maximum(
            cnta_ref[0, :N, 0:1] + cnta_ref[1, :N, 0:1], 1.0)
        cnt_ref[...] = cnt
        pre = (acc_ref[0, :N, :] + acc_ref[1, :N, :]) / cnt + z_ref[...]
        h_ref[...] = _bn_relu(pre, g_ref[...], b_ref[...])

    return pl.pallas_call(
        body,
        out_shape=[jax.ShapeDtypeStruct((N, fo), jnp.float32),
                   jax.ShapeDtypeStruct((N, 1), jnp.float32)],
    )(acc, cnta, z, g, b)


def _tc_combine_pre(acc, cnt, z, g, b, wl, wr, bl):
    fo = z.shape[1]
    fn = wl.shape[1]

    def body(acc_ref, cnt_ref, z_ref, g_ref, b_ref, wl_ref, wr_ref, bl_ref,
             h_ref, y_ref, z2_ref):
        pre = (acc_ref[0, :N, :] + acc_ref[1, :N, :]) / cnt_ref[...] + z_ref[...]
        h = _bn_relu(pre, g_ref[...], b_ref[...])
        h_ref[...] = h
        y_ref[...] = _dot(h, wl_ref[...])
        z2_ref[...] = _dot(h, wr_ref[...]) + bl_ref[...]

    return pl.pallas_call(
        body,
        out_shape=[jax.ShapeDtypeStruct((N, fo), jnp.float32),
                   jax.ShapeDtypeStruct((N, fn), jnp.float32),
                   jax.ShapeDtypeStruct((N, fn), jnp.float32)],
    )(acc, cnt, z, g, b, wl, wr, bl)


def _tc_final(acc, cnt, z, g, b, ws1, bs1, ws2, bs2):
    fo = z.shape[1]

    def body(acc_ref, cnt_ref, z_ref, g_ref, b_ref, ws1_ref, bs1_ref,
             ws2_ref, bs2_ref, h_ref, score_ref):
        pre = (acc_ref[0, :N, :] + acc_ref[1, :N, :]) / cnt_ref[...] + z_ref[...]
        h = _bn_relu(pre, g_ref[...], b_ref[...])
        h_ref[...] = h
        ge = jnp.mean(h, axis=0, keepdims=True)
        s = jnp.maximum(_dot(ge, ws1_ref[...]) + bs1_ref[...], 0.0)
        t = _dot(s, ws2_ref[...]) + bs2_ref[...]
        score_ref[...] = 1.0 / (1.0 + jnp.exp(-t))

    return pl.pallas_call(
        body,
        out_shape=[jax.ShapeDtypeStruct((N, fo), jnp.float32),
                   jax.ShapeDtypeStruct((1, 1), jnp.float32)],
    )(acc, cnt, z, g, b, ws1, bs1, ws2, bs2)


def kernel(x, edge_index, params):
    src = edge_index[0].astype(jnp.int32)
    dst = edge_index[1].astype(jnp.int32)
    pad = EPAD - E
    srcb = jnp.concatenate([src, jnp.zeros((pad,), jnp.int32)]).reshape(
        16, NPAIR, C)
    dstb = jnp.concatenate([dst, jnp.full((pad,), N, jnp.int32)]).reshape(
        16, NPAIR, C)
    zcnt = jnp.zeros((R, CW), jnp.float32)
    onesb = jnp.ones((C, CW), jnp.float32)

    p = params
    row = lambda v: v.reshape(1, -1)

    y, z = _tc_pre(x, p['Wl0'], p['Wr0'], row(p['bl0']))

    agg0 = _make_agg(DIMS[0], with_counts=True)
    acc, cnta = agg0(y, srcb, dstb, jnp.zeros((R, DIMS[0]), jnp.float32),
                     zcnt, onesb)
    h, cnt = _tc_combine_first(acc, cnta, z, row(p['bn_g0']), row(p['bn_b0']))
    y, z = _tc_pre(h, p['Wl1'], p['Wr1'], row(p['bl1']))

    for i in (1, 2):
        agg = _make_agg(DIMS[i], with_counts=False)
        (acc,) = agg(y, srcb, dstb, jnp.zeros((R, DIMS[i]), jnp.float32))
        h, y, z = _tc_combine_pre(
            acc, cnt, z, row(p[f'bn_g{i}']), row(p[f'bn_b{i}']),
            p[f'Wl{i + 1}'], p[f'Wr{i + 1}'], row(p[f'bl{i + 1}']))

    agg3 = _make_agg(DIMS[3], with_counts=False)
    (acc,) = agg3(y, srcb, dstb, jnp.zeros((R, DIMS[3]), jnp.float32))
    h, score = _tc_final(acc, cnt, z, row(p['bn_g3']), row(p['bn_b3']),
                         p['Ws1'], row(p['bs1']), p['Ws2'], row(p['bs2']))
    return (h, score)


# speedup vs baseline: 1.0511x; 1.0511x over previous
import functools

import jax
import jax.numpy as jnp
from jax import lax
from jax.experimental import pallas as pl
from jax.experimental.pallas import tpu as pltpu
from jax.experimental.pallas import tpu_sc as plsc

N = 10000
E = 320000
DIMS = [128, 64, 32, 16]
BN_EPS = 1e-5

NW = 32
C = 128
NPAIR = 160
EPAD = 16 * NPAIR * C
KSTAGE = 8
NC0S = {128: 120, 64: 112, 32: 104, 16: 88}
R = 10112
RPT = R // 16
CW = 8


def _make_agg(fo, with_counts):
    nc0 = NC0S[fo]
    mesh = plsc.VectorSubcoreMesh(
        core_axis_name="c", subcore_axis_name="s", num_cores=2, num_subcores=16
    )
    out_type = [jax.ShapeDtypeStruct((2, R, fo), jnp.float32)]
    scratch = [
        pltpu.VMEM_SHARED((R, fo), jnp.float32),
        pltpu.VMEM((KSTAGE, C), jnp.int32),
        pltpu.VMEM((KSTAGE, C), jnp.int32),
        pltpu.VMEM((C, fo), jnp.float32),
        pltpu.VMEM((C, fo), jnp.float32),
        pltpu.SemaphoreType.DMA,
        pltpu.SemaphoreType.DMA,
    ]
    if with_counts:
        out_type.append(jax.ShapeDtypeStruct((2, R, CW), jnp.float32))
        scratch += [
            pltpu.VMEM_SHARED((R, CW), jnp.float32),
            pltpu.VMEM((C, CW), jnp.float32),
        ]

    def body(table, srcb, dstb, zrows, *rest):
        if with_counts:
            (zcnt, onesb, outacc, outcnt, acc_sh, src_v, dst_v,
             rows0, rows1, gs0, gs1, cnt_sh, ones_v) = rest
        else:
            (outacc, acc_sh, src_v, dst_v,
             rows0, rows1, gs0, gs1) = rest
        rows = (rows0, rows1)
        gsem = (gs0, gs1)
        cid = lax.axis_index("c")
        sid = lax.axis_index("s")
        base = jnp.where(cid == 0, 0, nc0)
        nstage = jnp.where(cid == 0, nc0 // KSTAGE, (NPAIR - nc0) // KSTAGE)
        rs = pl.ds(sid * RPT, RPT)
        pltpu.sync_copy(zrows.at[rs], acc_sh.at[rs])
        if with_counts:
            pltpu.sync_copy(zcnt.at[rs], cnt_sh.at[rs])
            pltpu.sync_copy(onesb, ones_v)
        plsc.subcore_barrier()

        def stage(t, carry):
            pltpu.sync_copy(srcb.at[sid, pl.ds(base + t * KSTAGE, KSTAGE)], src_v)
            pltpu.sync_copy(dstb.at[sid, pl.ds(base + t * KSTAGE, KSTAGE)], dst_v)
            gd = [None, None]
            gd[0] = pltpu.async_copy(table.at[src_v.at[0]], rows[0], gsem[0])
            for j in range(KSTAGE):
                p = j % 2
                gd[p].wait()
                if j + 1 < KSTAGE:
                    gd[1 - p] = pltpu.async_copy(
                        table.at[src_v.at[j + 1]], rows[1 - p], gsem[1 - p])
                pltpu.sync_copy(rows[p], acc_sh.at[dst_v.at[j]], add=True)
                if with_counts:
                    pltpu.sync_copy(ones_v, cnt_sh.at[dst_v.at[j]], add=True)
            return carry

        lax.fori_loop(0, nstage, stage, 0)
        plsc.subcore_barrier()
        pltpu.sync_copy(acc_sh.at[rs], outacc.at[cid, rs])
        if with_counts:
            pltpu.sync_copy(cnt_sh.at[rs], outcnt.at[cid, rs])

    return pl.kernel(
        body, out_type=out_type, mesh=mesh, scratch_types=scratch,
        compiler_params=pltpu.CompilerParams(use_tc_tiling_on_sc=False))


def _dot(a, b):
    return jnp.dot(a, b, preferred_element_type=jnp.float32,
                   precision=lax.Precision.HIGHEST)


def _tc_pre(x, wl, wr, bl):
    def body(x_ref, wl_ref, wr_ref, bl_ref, y_ref, z_ref):
        xv = x_ref[...]
        y_ref[...] = _dot(xv, wl_ref[...])
        z_ref[...] = _dot(xv, wr_ref[...]) + bl_ref[...]

    fo = wl.shape[1]
    return pl.pallas_call(
        body,
        out_shape=[jax.ShapeDtypeStruct((N, fo), jnp.float32),
                   jax.ShapeDtypeStruct((N, fo), jnp.float32)],
    )(x, wl, wr, bl)


def _bn_relu(pre, g, b):
    mu = jnp.mean(pre, axis=0, keepdims=True)
    var = jnp.mean((pre - mu) ** 2, axis=0, keepdims=True)
    return jnp.maximum((pre - mu) * lax.rsqrt(var + BN_EPS) * g + b, 0.0)


def _tc_combine_first(acc, cnta, z, g, b):
    fo = z.shape[1]

    def body(acc_ref, cnta_ref, z_ref, g_ref, b_ref, h_ref, cnt_ref):
        cnt = jnp.maximum(
            cnta_ref[0, :N, 0:1] + cnta_ref[1, :N, 0:1], 1.0)
        cnt_ref[...] = cnt
        pre = (acc_ref[0, :N, :] + acc_ref[1, :N, :]) / cnt + z_ref[...]
        h_ref[...] = _bn_relu(pre, g_ref[...], b_ref[...])

    return pl.pallas_call(
        body,
        out_shape=[jax.ShapeDtypeStruct((N, fo), jnp.float32),
                   jax.ShapeDtypeStruct((N, 1), jnp.float32)],
    )(acc, cnta, z, g, b)


def _tc_combine_pre(acc, cnt, z, g, b, wl, wr, bl):
    fo = z.shape[1]
    fn = wl.shape[1]

    def body(acc_ref, cnt_ref, z_ref, g_ref, b_ref, wl_ref, wr_ref, bl_ref,
             h_ref, y_ref, z2_ref):
        pre = (acc_ref[0, :N, :] + acc_ref[1, :N, :]) / cnt_ref[...] + z_ref[...]
        h = _bn_relu(pre, g_ref[...], b_ref[...])
        h_ref[...] = h
        y_ref[...] = _dot(h, wl_ref[...])
        z2_ref[...] = _dot(h, wr_ref[...]) + bl_ref[...]

    return pl.pallas_call(
        body,
        out_shape=[jax.ShapeDtypeStruct((N, fo), jnp.float32),
                   jax.ShapeDtypeStruct((N, fn), jnp.float32),
                   jax.ShapeDtypeStruct((N, fn), jnp.float32)],
    )(acc, cnt, z, g, b, wl, wr, bl)


def _tc_final(acc, cnt, z, g, b, ws1, bs1, ws2, bs2):
    fo = z.shape[1]

    def body(acc_ref, cnt_ref, z_ref, g_ref, b_ref, ws1_ref, bs1_ref,
             ws2_ref, bs2_ref, h_ref, score_ref):
        pre = (acc_ref[0, :N, :] + acc_ref[1, :N, :]) / cnt_ref[...] + z_ref[...]
        h = _bn_relu(pre, g_ref[...], b_ref[...])
        h_ref[...] = h
        ge = jnp.mean(h, axis=0, keepdims=True)
        s = jnp.maximum(_dot(ge, ws1_ref[...]) + bs1_ref[...], 0.0)
        t = _dot(s, ws2_ref[...]) + bs2_ref[...]
        score_ref[...] = 1.0 / (1.0 + jnp.exp(-t))

    return pl.pallas_call(
        body,
        out_shape=[jax.ShapeDtypeStruct((N, fo), jnp.float32),
                   jax.ShapeDtypeStruct((1, 1), jnp.float32)],
    )(acc, cnt, z, g, b, ws1, bs1, ws2, bs2)


def kernel(x, edge_index, params):
    src = edge_index[0].astype(jnp.int32)
    dst = edge_index[1].astype(jnp.int32)
    pad = EPAD - E
    srcb = jnp.concatenate([src, jnp.zeros((pad,), jnp.int32)]).reshape(
        16, NPAIR, C)
    dstb = jnp.concatenate([dst, jnp.full((pad,), N, jnp.int32)]).reshape(
        16, NPAIR, C)
    zcnt = jnp.zeros((R, CW), jnp.float32)
    onesb = jnp.ones((C, CW), jnp.float32)

    p = params
    row = lambda v: v.reshape(1, -1)

    y, z = _tc_pre(x, p['Wl0'], p['Wr0'], row(p['bl0']))

    agg0 = _make_agg(DIMS[0], with_counts=True)
    acc, cnta = agg0(y, srcb, dstb, jnp.zeros((R, DIMS[0]), jnp.float32),
                     zcnt, onesb)
    h, cnt = _tc_combine_first(acc, cnta, z, row(p['bn_g0']), row(p['bn_b0']))
    y, z = _tc_pre(h, p['Wl1'], p['Wr1'], row(p['bl1']))

    for i in (1, 2):
        agg = _make_agg(DIMS[i], with_counts=False)
        (acc,) = agg(y, srcb, dstb, jnp.zeros((R, DIMS[i]), jnp.float32))
        h, y, z = _tc_combine_pre(
            acc, cnt, z, row(p[f'bn_g{i}']), row(p[f'bn_b{i}']),
            p[f'Wl{i + 1}'], p[f'Wr{i + 1}'], row(p[f'bl{i + 1}']))

    agg3 = _make_agg(DIMS[3], with_counts=False)
    (acc,) = agg3(y, srcb, dstb, jnp.zeros((R, DIMS[3]), jnp.float32))
    h, score = _tc_final(acc, cnt, z, row(p['bn_g3']), row(p['bn_b3']),
                         p['Ws1'], row(p['bs1']), p['Ws2'], row(p['bs2']))
    return (h, score)


# speedup vs baseline: 1.0536x; 1.0023x over previous
import functools

import jax
import jax.numpy as jnp
from jax import lax
from jax.experimental import pallas as pl
from jax.experimental.pallas import tpu as pltpu
from jax.experimental.pallas import tpu_sc as plsc

N = 10000
E = 320000
DIMS = [128, 64, 32, 16]
BN_EPS = 1e-5

NW = 32
C = 128
NPAIR = 160
EPAD = 16 * NPAIR * C
KSTAGE = 8
NC0S = {128: 120, 64: 112, 32: 104, 16: 88}
R = 10112
RPT = R // 16
CW = 8


def _make_agg(fo, with_counts):
    nc0 = NC0S[fo]
    mesh = plsc.VectorSubcoreMesh(
        core_axis_name="c", subcore_axis_name="s", num_cores=2, num_subcores=16
    )
    out_type = [jax.ShapeDtypeStruct((2, R, fo), jnp.float32)]
    scratch = [
        pltpu.VMEM_SHARED((R, fo), jnp.float32),
        pltpu.VMEM((KSTAGE, C), jnp.int32),
        pltpu.VMEM((KSTAGE, C), jnp.int32),
        pltpu.VMEM((C, fo), jnp.float32),
        pltpu.VMEM((C, fo), jnp.float32),
        pltpu.SemaphoreType.DMA,
        pltpu.SemaphoreType.DMA,
    ]
    if with_counts:
        out_type.append(jax.ShapeDtypeStruct((2, R, CW), jnp.float32))
        scratch += [
            pltpu.VMEM_SHARED((R, CW), jnp.float32),
            pltpu.VMEM((C, CW), jnp.float32),
        ]

    def body(table, srcb, dstb, *rest):
        if with_counts:
            (zcnt, onesb, outacc, outcnt, acc_sh, src_v, dst_v,
             rows0, rows1, gs0, gs1, cnt_sh, ones_v) = rest
        else:
            (outacc, acc_sh, src_v, dst_v,
             rows0, rows1, gs0, gs1) = rest
        rows = (rows0, rows1)
        gsem = (gs0, gs1)
        cid = lax.axis_index("c")
        sid = lax.axis_index("s")
        base = jnp.where(cid == 0, 0, nc0)
        nstage = jnp.where(cid == 0, nc0 // KSTAGE, (NPAIR - nc0) // KSTAGE)
        rs = pl.ds(sid * RPT, RPT)
        z16 = jnp.zeros((16,), jnp.float32)
        for r in range(C):
            for c in range(fo // 16):
                rows0[r, pl.ds(c * 16, 16)] = z16
        for q in range(RPT // C):
            pltpu.sync_copy(rows0, acc_sh.at[pl.ds(sid * RPT + q * C, C)])
        rem = RPT - (RPT // C) * C
        if rem:
            pltpu.sync_copy(rows0.at[pl.ds(0, rem)],
                            acc_sh.at[pl.ds(sid * RPT + (RPT // C) * C, rem)])
        if with_counts:
            pltpu.sync_copy(zcnt.at[rs], cnt_sh.at[rs])
            pltpu.sync_copy(onesb, ones_v)
        plsc.subcore_barrier()

        def stage(t, carry):
            pltpu.sync_copy(srcb.at[sid, pl.ds(base + t * KSTAGE, KSTAGE)], src_v)
            pltpu.sync_copy(dstb.at[sid, pl.ds(base + t * KSTAGE, KSTAGE)], dst_v)
            gd = [None, None]
            gd[0] = pltpu.async_copy(table.at[src_v.at[0]], rows[0], gsem[0])
            for j in range(KSTAGE):
                p = j % 2
                gd[p].wait()
                if j + 1 < KSTAGE:
                    gd[1 - p] = pltpu.async_copy(
                        table.at[src_v.at[j + 1]], rows[1 - p], gsem[1 - p])
                pltpu.sync_copy(rows[p], acc_sh.at[dst_v.at[j]], add=True)
                if with_counts:
                    pltpu.sync_copy(ones_v, cnt_sh.at[dst_v.at[j]], add=True)
            return carry

        lax.fori_loop(0, nstage, stage, 0)
        plsc.subcore_barrier()
        pltpu.sync_copy(acc_sh.at[rs], outacc.at[cid, rs])
        if with_counts:
            pltpu.sync_copy(cnt_sh.at[rs], outcnt.at[cid, rs])

    return pl.kernel(
        body, out_type=out_type, mesh=mesh, scratch_types=scratch,
        compiler_params=pltpu.CompilerParams(use_tc_tiling_on_sc=False))


def _dot(a, b):
    return jnp.dot(a, b, preferred_element_type=jnp.float32,
                   precision=lax.Precision.HIGHEST)


def _tc_pre(x, wl, wr, bl):
    def body(x_ref, wl_ref, wr_ref, bl_ref, y_ref, z_ref):
        xv = x_ref[...]
        y_ref[...] = _dot(xv, wl_ref[...])
        z_ref[...] = _dot(xv, wr_ref[...]) + bl_ref[...]

    fo = wl.shape[1]
    return pl.pallas_call(
        body,
        out_shape=[jax.ShapeDtypeStruct((N, fo), jnp.float32),
                   jax.ShapeDtypeStruct((N, fo), jnp.float32)],
    )(x, wl, wr, bl)


def _bn_relu(pre, g, b):
    mu = jnp.mean(pre, axis=0, keepdims=True)
    var = jnp.mean((pre - mu) ** 2, axis=0, keepdims=True)
    return jnp.maximum((pre - mu) * lax.rsqrt(var + BN_EPS) * g + b, 0.0)


def _tc_combine_first(acc, cnta, z, g, b):
    fo = z.shape[1]

    def body(acc_ref, cnta_ref, z_ref, g_ref, b_ref, h_ref, cnt_ref):
        cnt = jnp.maximum(
            cnta_ref[0, :N, 0:1] + cnta_ref[1, :N, 0:1], 1.0)
        cnt_ref[...] = cnt
        pre = (acc_ref[0, :N, :] + acc_ref[1, :N, :]) / cnt + z_ref[...]
        h_ref[...] = _bn_relu(pre, g_ref[...], b_ref[...])

    return pl.pallas_call(
        body,
        out_shape=[jax.ShapeDtypeStruct((N, fo), jnp.float32),
                   jax.ShapeDtypeStruct((N, 1), jnp.float32)],
    )(acc, cnta, z, g, b)


def _tc_combine_pre(acc, cnt, z, g, b, wl, wr, bl):
    fo = z.shape[1]
    fn = wl.shape[1]

    def body(acc_ref, cnt_ref, z_ref, g_ref, b_ref, wl_ref, wr_ref, bl_ref,
             h_ref, y_ref, z2_ref):
        pre = (acc_ref[0, :N, :] + acc_ref[1, :N, :]) / cnt_ref[...] + z_ref[...]
        h = _bn_relu(pre, g_ref[...], b_ref[...])
        h_ref[...] = h
        y_ref[...] = _dot(h, wl_ref[...])
        z2_ref[...] = _dot(h, wr_ref[...]) + bl_ref[...]

    return pl.pallas_call(
        body,
        out_shape=[jax.ShapeDtypeStruct((N, fo), jnp.float32),
                   jax.ShapeDtypeStruct((N, fn), jnp.float32),
                   jax.ShapeDtypeStruct((N, fn), jnp.float32)],
    )(acc, cnt, z, g, b, wl, wr, bl)


def _tc_final(acc, cnt, z, g, b, ws1, bs1, ws2, bs2):
    fo = z.shape[1]

    def body(acc_ref, cnt_ref, z_ref, g_ref, b_ref, ws1_ref, bs1_ref,
             ws2_ref, bs2_ref, h_ref, score_ref):
        pre = (acc_ref[0, :N, :] + acc_ref[1, :N, :]) / cnt_ref[...] + z_ref[...]
        h = _bn_relu(pre, g_ref[...], b_ref[...])
        h_ref[...] = h
        ge = jnp.mean(h, axis=0, keepdims=True)
        s = jnp.maximum(_dot(ge, ws1_ref[...]) + bs1_ref[...], 0.0)
        t = _dot(s, ws2_ref[...]) + bs2_ref[...]
        score_ref[...] = 1.0 / (1.0 + jnp.exp(-t))

    return pl.pallas_call(
        body,
        out_shape=[jax.ShapeDtypeStruct((N, fo), jnp.float32),
                   jax.ShapeDtypeStruct((1, 1), jnp.float32)],
    )(acc, cnt, z, g, b, ws1, bs1, ws2, bs2)


def kernel(x, edge_index, params):
    src = edge_index[0].astype(jnp.int32)
    dst = edge_index[1].astype(jnp.int32)
    pad = EPAD - E
    srcb = jnp.concatenate([src, jnp.zeros((pad,), jnp.int32)]).reshape(
        16, NPAIR, C)
    dstb = jnp.concatenate([dst, jnp.full((pad,), N, jnp.int32)]).reshape(
        16, NPAIR, C)
    zcnt = jnp.zeros((R, CW), jnp.float32)
    onesb = jnp.ones((C, CW), jnp.float32)

    p = params
    row = lambda v: v.reshape(1, -1)

    y, z = _tc_pre(x, p['Wl0'], p['Wr0'], row(p['bl0']))

    agg0 = _make_agg(DIMS[0], with_counts=True)
    acc, cnta = agg0(y, srcb, dstb, zcnt, onesb)
    h, cnt = _tc_combine_first(acc, cnta, z, row(p['bn_g0']), row(p['bn_b0']))
    y, z = _tc_pre(h, p['Wl1'], p['Wr1'], row(p['bl1']))

    for i in (1, 2):
        agg = _make_agg(DIMS[i], with_counts=False)
        (acc,) = agg(y, srcb, dstb)
        h, y, z = _tc_combine_pre(
            acc, cnt, z, row(p[f'bn_g{i}']), row(p[f'bn_b{i}']),
            p[f'Wl{i + 1}'], p[f'Wr{i + 1}'], row(p[f'bl{i + 1}']))

    agg3 = _make_agg(DIMS[3], with_counts=False)
    (acc,) = agg3(y, srcb, dstb)
    h, score = _tc_final(acc, cnt, z, row(p['bn_g3']), row(p['bn_b3']),
                         p['Ws1'], row(p['bs1']), p['Ws2'], row(p['bs2']))
    return (h, score)


# speedup vs baseline: 1.1125x; 1.0560x over previous
import functools

import jax
import jax.numpy as jnp
from jax import lax
from jax.experimental import pallas as pl
from jax.experimental.pallas import tpu as pltpu
from jax.experimental.pallas import tpu_sc as plsc

N = 10000
E = 320000
DIMS = [128, 64, 32, 16]
BN_EPS = 1e-5

NW = 32
C = 128
NPAIR = 160
EPAD = 16 * NPAIR * C
KSTAGE = 8
NC0S = {128: 120, 64: 112, 32: 104, 16: 88}
R = 10112
RPT = R // 16
CW = 8


def _make_agg(fo, with_counts):
    nc0 = NC0S[fo]
    mesh = plsc.VectorSubcoreMesh(
        core_axis_name="c", subcore_axis_name="s", num_cores=2, num_subcores=16
    )
    nidx = KSTAGE if with_counts else max(nc0, NPAIR - nc0)
    out_type = [jax.ShapeDtypeStruct((2, R, fo), jnp.float32)]
    scratch = [
        pltpu.VMEM_SHARED((R, fo), jnp.float32),
        pltpu.VMEM((nidx, C), jnp.int32),
        pltpu.VMEM((nidx, C), jnp.int32),
        pltpu.VMEM((C, fo), jnp.float32),
        pltpu.VMEM((C, fo), jnp.float32),
        pltpu.SemaphoreType.DMA,
        pltpu.SemaphoreType.DMA,
    ]
    if with_counts:
        out_type.append(jax.ShapeDtypeStruct((2, R, CW), jnp.float32))
        scratch += [
            pltpu.VMEM_SHARED((R, CW), jnp.float32),
            pltpu.VMEM((C, CW), jnp.float32),
        ]

    def body(table, srcb, dstb, *rest):
        if with_counts:
            (zcnt, onesb, outacc, outcnt, acc_sh, src_v, dst_v,
             rows0, rows1, gs0, gs1, cnt_sh, ones_v) = rest
        else:
            (outacc, acc_sh, src_v, dst_v,
             rows0, rows1, gs0, gs1) = rest
        rows = (rows0, rows1)
        gsem = (gs0, gs1)
        cid = lax.axis_index("c")
        sid = lax.axis_index("s")
        base = jnp.where(cid == 0, 0, nc0)
        nstage = jnp.where(cid == 0, nc0 // KSTAGE, (NPAIR - nc0) // KSTAGE)
        rs = pl.ds(sid * RPT, RPT)
        z16 = jnp.zeros((16,), jnp.float32)
        for r in range(C):
            for c in range(fo // 16):
                rows0[r, pl.ds(c * 16, 16)] = z16
        for q in range(RPT // C):
            pltpu.sync_copy(rows0, acc_sh.at[pl.ds(sid * RPT + q * C, C)])
        rem = RPT - (RPT // C) * C
        if rem:
            pltpu.sync_copy(rows0.at[pl.ds(0, rem)],
                            acc_sh.at[pl.ds(sid * RPT + (RPT // C) * C, rem)])
        if with_counts:
            pltpu.sync_copy(zcnt.at[rs], cnt_sh.at[rs])
            pltpu.sync_copy(onesb, ones_v)
        plsc.subcore_barrier()

        if with_counts:
            def stage(t, carry):
                pltpu.sync_copy(
                    srcb.at[sid, pl.ds(base + t * KSTAGE, KSTAGE)], src_v)
                pltpu.sync_copy(
                    dstb.at[sid, pl.ds(base + t * KSTAGE, KSTAGE)], dst_v)
                gd = [None, None]
                gd[0] = pltpu.async_copy(table.at[src_v.at[0]], rows[0], gsem[0])
                for j in range(KSTAGE):
                    p = j % 2
                    gd[p].wait()
                    if j + 1 < KSTAGE:
                        gd[1 - p] = pltpu.async_copy(
                            table.at[src_v.at[j + 1]], rows[1 - p], gsem[1 - p])
                    pltpu.sync_copy(rows[p], acc_sh.at[dst_v.at[j]], add=True)
                    pltpu.sync_copy(ones_v, cnt_sh.at[dst_v.at[j]], add=True)
                return carry

            lax.fori_loop(0, nstage, stage, 0)
        else:
            nch = jnp.where(cid == 0, nc0, NPAIR - nc0)
            bload = jnp.minimum(base, NPAIR - nidx)
            off = base - bload
            pltpu.sync_copy(srcb.at[sid, pl.ds(bload, nidx)], src_v)
            pltpu.sync_copy(dstb.at[sid, pl.ds(bload, nidx)], dst_v)

            def gwait(p):
                pltpu.make_async_copy(
                    table.at[src_v.at[0]], rows[p], gsem[p]).wait()

            pltpu.async_copy(table.at[src_v.at[off]], rows[0], gsem[0])
            pltpu.async_copy(table.at[src_v.at[off + 1]], rows[1], gsem[1])

            def step(t, carry):
                for p in (0, 1):
                    g = 2 * t + p
                    gwait(p)
                    pltpu.sync_copy(rows[p], acc_sh.at[dst_v.at[off + g]],
                                    add=True)
                    nxt = jnp.minimum(g + 2, nch - 1)
                    pltpu.async_copy(
                        table.at[src_v.at[off + nxt]], rows[p], gsem[p])
                return carry

            lax.fori_loop(0, lax.div(nch, 2), step, 0)
            gwait(0)
            gwait(1)
        plsc.subcore_barrier()
        pltpu.sync_copy(acc_sh.at[rs], outacc.at[cid, rs])
        if with_counts:
            pltpu.sync_copy(cnt_sh.at[rs], outcnt.at[cid, rs])

    return pl.kernel(
        body, out_type=out_type, mesh=mesh, scratch_types=scratch,
        compiler_params=pltpu.CompilerParams(use_tc_tiling_on_sc=False))


def _dot(a, b):
    return jnp.dot(a, b, preferred_element_type=jnp.float32,
                   precision=lax.Precision.HIGHEST)


def _tc_pre(x, wl, wr, bl):
    def body(x_ref, wl_ref, wr_ref, bl_ref, y_ref, z_ref):
        xv = x_ref[...]
        y_ref[...] = _dot(xv, wl_ref[...])
        z_ref[...] = _dot(xv, wr_ref[...]) + bl_ref[...]

    fo = wl.shape[1]
    return pl.pallas_call(
        body,
        out_shape=[jax.ShapeDtypeStruct((N, fo), jnp.float32),
                   jax.ShapeDtypeStruct((N, fo), jnp.float32)],
    )(x, wl, wr, bl)


def _bn_relu(pre, g, b):
    mu = jnp.mean(pre, axis=0, keepdims=True)
    var = jnp.mean((pre - mu) ** 2, axis=0, keepdims=True)
    return jnp.maximum((pre - mu) * lax.rsqrt(var + BN_EPS) * g + b, 0.0)


def _tc_combine_first(acc, cnta, z, g, b):
    fo = z.shape[1]

    def body(acc_ref, cnta_ref, z_ref, g_ref, b_ref, h_ref, cnt_ref):
        cnt = jnp.maximum(
            cnta_ref[0, :N, 0:1] + cnta_ref[1, :N, 0:1], 1.0)
        cnt_ref[...] = cnt
        pre = (acc_ref[0, :N, :] + acc_ref[1, :N, :]) / cnt + z_ref[...]
        h_ref[...] = _bn_relu(pre, g_ref[...], b_ref[...])

    return pl.pallas_call(
        body,
        out_shape=[jax.ShapeDtypeStruct((N, fo), jnp.float32),
                   jax.ShapeDtypeStruct((N, 1), jnp.float32)],
    )(acc, cnta, z, g, b)


def _tc_combine_pre(acc, cnt, z, g, b, wl, wr, bl):
    fo = z.shape[1]
    fn = wl.shape[1]

    def body(acc_ref, cnt_ref, z_ref, g_ref, b_ref, wl_ref, wr_ref, bl_ref,
             h_ref, y_ref, z2_ref):
        pre = (acc_ref[0, :N, :] + acc_ref[1, :N, :]) / cnt_ref[...] + z_ref[...]
        h = _bn_relu(pre, g_ref[...], b_ref[...])
        h_ref[...] = h
        y_ref[...] = _dot(h, wl_ref[...])
        z2_ref[...] = _dot(h, wr_ref[...]) + bl_ref[...]

    return pl.pallas_call(
        body,
        out_shape=[jax.ShapeDtypeStruct((N, fo), jnp.float32),
                   jax.ShapeDtypeStruct((N, fn), jnp.float32),
                   jax.ShapeDtypeStruct((N, fn), jnp.float32)],
    )(acc, cnt, z, g, b, wl, wr, bl)


def _tc_final(acc, cnt, z, g, b, ws1, bs1, ws2, bs2):
    fo = z.shape[1]

    def body(acc_ref, cnt_ref, z_ref, g_ref, b_ref, ws1_ref, bs1_ref,
             ws2_ref, bs2_ref, h_ref, score_ref):
        pre = (acc_ref[0, :N, :] + acc_ref[1, :N, :]) / cnt_ref[...] + z_ref[...]
        h = _bn_relu(pre, g_ref[...], b_ref[...])
        h_ref[...] = h
        ge = jnp.mean(h, axis=0, keepdims=True)
        s = jnp.maximum(_dot(ge, ws1_ref[...]) + bs1_ref[...], 0.0)
        t = _dot(s, ws2_ref[...]) + bs2_ref[...]
        score_ref[...] = 1.0 / (1.0 + jnp.exp(-t))

    return pl.pallas_call(
        body,
        out_shape=[jax.ShapeDtypeStruct((N, fo), jnp.float32),
                   jax.ShapeDtypeStruct((1, 1), jnp.float32)],
    )(acc, cnt, z, g, b, ws1, bs1, ws2, bs2)


def kernel(x, edge_index, params):
    src = edge_index[0].astype(jnp.int32)
    dst = edge_index[1].astype(jnp.int32)
    pad = EPAD - E
    srcb = jnp.concatenate([src, jnp.zeros((pad,), jnp.int32)]).reshape(
        16, NPAIR, C)
    dstb = jnp.concatenate([dst, jnp.full((pad,), N, jnp.int32)]).reshape(
        16, NPAIR, C)
    zcnt = jnp.zeros((R, CW), jnp.float32)
    onesb = jnp.ones((C, CW), jnp.float32)

    p = params
    row = lambda v: v.reshape(1, -1)

    y, z = _tc_pre(x, p['Wl0'], p['Wr0'], row(p['bl0']))

    agg0 = _make_agg(DIMS[0], with_counts=True)
    acc, cnta = agg0(y, srcb, dstb, zcnt, onesb)
    h, cnt = _tc_combine_first(acc, cnta, z, row(p['bn_g0']), row(p['bn_b0']))
    y, z = _tc_pre(h, p['Wl1'], p['Wr1'], row(p['bl1']))

    for i in (1, 2):
        agg = _make_agg(DIMS[i], with_counts=False)
        (acc,) = agg(y, srcb, dstb)
        h, y, z = _tc_combine_pre(
            acc, cnt, z, row(p[f'bn_g{i}']), row(p[f'bn_b{i}']),
            p[f'Wl{i + 1}'], p[f'Wr{i + 1}'], row(p[f'bl{i + 1}']))

    agg3 = _make_agg(DIMS[3], with_counts=False)
    (acc,) = agg3(y, srcb, dstb)
    h, score = _tc_final(acc, cnt, z, row(p['bn_g3']), row(p['bn_b3']),
                         p['Ws1'], row(p['bs1']), p['Ws2'], row(p['bs2']))
    return (h, score)
